# baseline (speedup 1.0000x reference)
import functools

import jax
import jax.numpy as jnp
from jax import lax
from jax.experimental import pallas as pl
from jax.experimental.pallas import tpu as pltpu
from jax.experimental.pallas import tpu_sc as plsc

_NC = 2
_NS = 16

_CHUNK = 4096


def _gather_transposed(emb_t, u, v):
    D, V = emb_t.shape
    B = u.shape[0]
    n_chunks = B // _CHUNK

    mesh = plsc.VectorSubcoreMesh(core_axis_name="c", subcore_axis_name="s")

    @functools.partial(
        pl.kernel,
        mesh=mesh,
        compiler_params=pltpu.CompilerParams(needs_layout_passes=False),
        out_type=jax.ShapeDtypeStruct((2 * D, B), jnp.float32),
        scratch_types=[
            pltpu.VMEM((V,), jnp.float32),
            pltpu.VMEM((_CHUNK,), jnp.int32),
            pltpu.VMEM((_CHUNK,), jnp.int32),
            pltpu.VMEM((_CHUNK,), jnp.float32),
            pltpu.VMEM((_CHUNK,), jnp.float32),
            pltpu.SemaphoreType.DMA,
            pltpu.SemaphoreType.DMA,
            pltpu.SemaphoreType.DMA,
            pltpu.SemaphoreType.DMA,
        ],
    )
    def gather_k(emb_hbm, u_hbm, v_hbm, xt_hbm, feat, idxb0, idxb1,
                 outb0, outb1, si0, si1, so0, so1):
        wid = lax.axis_index("s") * _NC + lax.axis_index("c")
        idxb = (idxb0, idxb1)
        outb = (outb0, outb1)
        si = (si0, si1)
        so = (so0, so1)
        tasks = [(u_hbm, wid, c) for c in range(n_chunks)]
        tasks += [(v_hbm, wid + D, c) for c in range(n_chunks)]
        nt = len(tasks)

        def start_idx(t):
            idx_hbm, _, c = tasks[t]
            slot = t % 2
            return pltpu.async_copy(
                idx_hbm.at[pl.ds(c * _CHUNK, _CHUNK)], idxb[slot], si[slot])

        h_idx = {0: start_idx(0)}
        with jax.named_scope("feat_stage"):
            pltpu.sync_copy(emb_hbm.at[wid], feat)
        h_out = {}
        for t in range(nt):
            slot = t % 2
            h_idx[t].wait()
            if t + 1 < nt:
                h_idx[t + 1] = start_idx(t + 1)
            if t >= 2:
                h_out[t - 2].wait()
            src = idxb[slot]
            dst = outb[slot]

            with jax.named_scope("gather"):
                @plsc.parallel_loop(0, _CHUNK // 16, unroll=16)
                def _(i):
                    iv = src[pl.ds(i * 16, 16)]
                    dst[pl.ds(i * 16, 16)] = plsc.load_gather(feat, [iv])

            _, row, c = tasks[t]
            h_out[t] = pltpu.async_copy(
                outb[slot], xt_hbm.at[row, pl.ds(c * _CHUNK, _CHUNK)],
                so[slot])
        h_out[nt - 2].wait()
        h_out[nt - 1].wait()

    return gather_k(emb_t, u, v)


def _mlp_t(xt, w1, b1, w2, b2, w3, b3):
    D2, B = xt.shape
    blk = 4096
    dot = functools.partial(
        lax.dot_general, preferred_element_type=jnp.float32)
    dims = (((1,), (0,)), ((), ()))

    def body(xt_ref, w1_ref, b1_ref, w2_ref, b2_ref, w3_ref, b3_ref, o_ref):
        h = jnp.maximum(dot(w1_ref[...], xt_ref[...], dims) + b1_ref[...], 0.0)
        h = jnp.maximum(dot(w2_ref[...], h, dims) + b2_ref[...], 0.0)
        o_ref[...] = dot(w3_ref[...], h, dims) + b3_ref[...]

    return pl.pallas_call(
        body,
        grid=(B // blk,),
        in_specs=[
            pl.BlockSpec((D2, blk), lambda i: (0, i)),
            pl.BlockSpec((128, D2), lambda i: (0, 0)),
            pl.BlockSpec((128, 1), lambda i: (0, 0)),
            pl.BlockSpec((64, 128), lambda i: (0, 0)),
            pl.BlockSpec((64, 1), lambda i: (0, 0)),
            pl.BlockSpec((1, 64), lambda i: (0, 0)),
            pl.BlockSpec((1, 1), lambda i: (0, 0)),
        ],
        out_specs=pl.BlockSpec((1, blk), lambda i: (0, i)),
        out_shape=jax.ShapeDtypeStruct((1, B), jnp.float32),
    )(xt, w1, b1, w2, b2, w3, b3)


def kernel(u, v, emb, W1, b1, W2, b2, W3, b3):
    u = u.astype(jnp.int32)
    v = v.astype(jnp.int32)
    xt = _gather_transposed(emb.T, u, v)
    out_t = _mlp_t(xt, W1, b1.reshape(-1, 1), W2, b2.reshape(-1, 1),
                   W3, b3.reshape(-1, 1))
    return out_t.reshape(-1, 1)


# speedup vs baseline: 1.0771x; 1.0771x over previous
import functools

import jax
import jax.numpy as jnp
from jax import lax
from jax.experimental import pallas as pl
from jax.experimental.pallas import tpu as pltpu
from jax.experimental.pallas import tpu_sc as plsc

_NC = 2
_NS = 16

_CHUNK = 4096


def _gather_transposed(emb_t, u, v):
    D, V = emb_t.shape
    B = u.shape[0]
    n_chunks = B // _CHUNK

    mesh = plsc.VectorSubcoreMesh(core_axis_name="c", subcore_axis_name="s")

    @functools.partial(
        pl.kernel,
        mesh=mesh,
        compiler_params=pltpu.CompilerParams(needs_layout_passes=False),
        out_type=jax.ShapeDtypeStruct((2 * D, B), jnp.float32),
        scratch_types=[
            pltpu.VMEM((V,), jnp.float32),
            pltpu.VMEM((_CHUNK,), jnp.int32),
            pltpu.VMEM((_CHUNK,), jnp.int32),
            pltpu.VMEM((_CHUNK,), jnp.int32),
            pltpu.VMEM((_CHUNK,), jnp.int32),
            pltpu.VMEM((_CHUNK,), jnp.float32),
            pltpu.VMEM((_CHUNK,), jnp.float32),
            pltpu.SemaphoreType.DMA,
            pltpu.SemaphoreType.DMA,
            pltpu.SemaphoreType.DMA,
            pltpu.SemaphoreType.DMA,
            pltpu.SemaphoreType.DMA,
            pltpu.SemaphoreType.DMA,
        ],
    )
    def gather_k(emb_hbm, u_hbm, v_hbm, xt_hbm, feat,
                 idxb0, idxb1, idxb2, idxb3, outb0, outb1,
                 si0, si1, si2, si3, so0, so1):
        wid = lax.axis_index("s") * _NC + lax.axis_index("c")
        idxb = (idxb0, idxb1, idxb2, idxb3)
        outb = (outb0, outb1)
        si = (si0, si1, si2, si3)
        so = (so0, so1)
        tasks = [(u_hbm, wid, c) for c in range(n_chunks)]
        tasks += [(v_hbm, wid + D, c) for c in range(n_chunks)]
        nt = len(tasks)

        def start_idx(t):
            idx_hbm, _, c = tasks[t]
            slot = t % 4
            return pltpu.async_copy(
                idx_hbm.at[pl.ds(c * _CHUNK, _CHUNK)], idxb[slot], si[slot])

        h_idx = {t: start_idx(t) for t in range(min(4, nt))}
        with jax.named_scope("feat_stage"):
            pltpu.sync_copy(emb_hbm.at[wid], feat)
        h_out = {}
        for t in range(nt):
            islot = t % 4
            oslot = t % 2
            h_idx[t].wait()
            if t + 4 < nt:
                h_idx[t + 4] = start_idx(t + 4)
            if t >= 2:
                h_out[t - 2].wait()
            src = idxb[islot]
            dst = outb[oslot]

            with jax.named_scope("gather"):
                @plsc.parallel_loop(0, _CHUNK // 16, unroll=16)
                def _(i):
                    iv = src[pl.ds(i * 16, 16)]
                    dst[pl.ds(i * 16, 16)] = plsc.load_gather(feat, [iv])

            _, row, c = tasks[t]
            h_out[t] = pltpu.async_copy(
                outb[oslot], xt_hbm.at[row, pl.ds(c * _CHUNK, _CHUNK)],
                so[oslot])
        h_out[nt - 2].wait()
        h_out[nt - 1].wait()

    return gather_k(emb_t, u, v)


def _mlp_t(xt, w1, b1, w2, b2, w3, b3):
    D2, B = xt.shape
    blk = 4096
    dot = functools.partial(
        lax.dot_general, preferred_element_type=jnp.float32)
    dims = (((1,), (0,)), ((), ()))

    def body(xt_ref, w1_ref, b1_ref, w2_ref, b2_ref, w3_ref, b3_ref, o_ref):
        h = jnp.maximum(dot(w1_ref[...], xt_ref[...], dims) + b1_ref[...], 0.0)
        h = jnp.maximum(dot(w2_ref[...], h, dims) + b2_ref[...], 0.0)
        o_ref[...] = dot(w3_ref[...], h, dims) + b3_ref[...]

    return pl.pallas_call(
        body,
        grid=(B // blk,),
        in_specs=[
            pl.BlockSpec((D2, blk), lambda i: (0, i)),
            pl.BlockSpec((128, D2), lambda i: (0, 0)),
            pl.BlockSpec((128, 1), lambda i: (0, 0)),
            pl.BlockSpec((64, 128), lambda i: (0, 0)),
            pl.BlockSpec((64, 1), lambda i: (0, 0)),
            pl.BlockSpec((1, 64), lambda i: (0, 0)),
            pl.BlockSpec((1, 1), lambda i: (0, 0)),
        ],
        out_specs=pl.BlockSpec((1, blk), lambda i: (0, i)),
        out_shape=jax.ShapeDtypeStruct((1, B), jnp.float32),
    )(xt, w1, b1, w2, b2, w3, b3)


def kernel(u, v, emb, W1, b1, W2, b2, W3, b3):
    u = u.astype(jnp.int32)
    v = v.astype(jnp.int32)
    xt = _gather_transposed(emb.T, u, v)
    out_t = _mlp_t(xt, W1, b1.reshape(-1, 1), W2, b2.reshape(-1, 1),
                   W3, b3.reshape(-1, 1))
    return out_t.reshape(-1, 1)


# speedup vs baseline: 1.0828x; 1.0053x over previous
import functools

import jax
import jax.numpy as jnp
from jax import lax
from jax.experimental import pallas as pl
from jax.experimental.pallas import tpu as pltpu
from jax.experimental.pallas import tpu_sc as plsc

_NC = 2
_NS = 16

_CHUNK = 4096


def _gather_transposed(emb_t, u, v):
    D, V = emb_t.shape
    B = u.shape[0]
    n_chunks = B // _CHUNK

    mesh = plsc.VectorSubcoreMesh(core_axis_name="c", subcore_axis_name="s")

    @functools.partial(
        pl.kernel,
        mesh=mesh,
        compiler_params=pltpu.CompilerParams(needs_layout_passes=False),
        out_type=jax.ShapeDtypeStruct((2 * D, B), jnp.float32),
        scratch_types=[
            pltpu.VMEM((V,), jnp.float32),
            pltpu.VMEM((_CHUNK,), jnp.int32),
            pltpu.VMEM((_CHUNK,), jnp.int32),
            pltpu.VMEM((_CHUNK,), jnp.int32),
            pltpu.VMEM((_CHUNK,), jnp.int32),
            pltpu.VMEM((_CHUNK,), jnp.float32),
            pltpu.VMEM((_CHUNK,), jnp.float32),
            pltpu.VMEM((_CHUNK,), jnp.float32),
            pltpu.SemaphoreType.DMA,
            pltpu.SemaphoreType.DMA,
            pltpu.SemaphoreType.DMA,
            pltpu.SemaphoreType.DMA,
            pltpu.SemaphoreType.DMA,
            pltpu.SemaphoreType.DMA,
            pltpu.SemaphoreType.DMA,
        ],
    )
    def gather_k(emb_hbm, u_hbm, v_hbm, xt_hbm, feat,
                 idxb0, idxb1, idxb2, idxb3, outb0, outb1, outb2,
                 si0, si1, si2, si3, so0, so1, so2):
        wid = lax.axis_index("s") * _NC + lax.axis_index("c")
        idxb = (idxb0, idxb1, idxb2, idxb3)
        outb = (outb0, outb1, outb2)
        si = (si0, si1, si2, si3)
        so = (so0, so1, so2)
        tasks = [(u_hbm, wid, c) for c in range(n_chunks)]
        tasks += [(v_hbm, wid + D, c) for c in range(n_chunks)]
        nt = len(tasks)

        def start_idx(t):
            idx_hbm, _, c = tasks[t]
            slot = t % 4
            return pltpu.async_copy(
                idx_hbm.at[pl.ds(c * _CHUNK, _CHUNK)], idxb[slot], si[slot])

        h_idx = {t: start_idx(t) for t in range(min(4, nt))}
        with jax.named_scope("feat_stage"):
            pltpu.sync_copy(emb_hbm.at[wid], feat)
        h_out = {}
        for t in range(nt):
            islot = t % 4
            oslot = t % 3
            h_idx[t].wait()
            if t + 4 < nt:
                h_idx[t + 4] = start_idx(t + 4)
            if t >= 3:
                h_out[t - 3].wait()
            src = idxb[islot]
            dst = outb[oslot]

            with jax.named_scope("gather"):
                @plsc.parallel_loop(0, _CHUNK // 16, unroll=16)
                def _(i):
                    iv = src[pl.ds(i * 16, 16)]
                    dst[pl.ds(i * 16, 16)] = plsc.load_gather(feat, [iv])

            _, row, c = tasks[t]
            h_out[t] = pltpu.async_copy(
                outb[oslot], xt_hbm.at[row, pl.ds(c * _CHUNK, _CHUNK)],
                so[oslot])
        h_out[nt - 3].wait()
        h_out[nt - 2].wait()
        h_out[nt - 1].wait()

    return gather_k(emb_t, u, v)


def _mlp_t(xt, w1, b1, w2, b2, w3, b3):
    D2, B = xt.shape
    blk = 4096
    dot = functools.partial(
        lax.dot_general, preferred_element_type=jnp.float32)
    dims = (((1,), (0,)), ((), ()))

    bf16 = jnp.bfloat16

    def body(xt_ref, w1_ref, b1_ref, w2_ref, b2_ref, w3_ref, b3_ref, o_ref):
        x = xt_ref[...].astype(bf16)
        h = jnp.maximum(
            dot(w1_ref[...].astype(bf16), x, dims) + b1_ref[...], 0.0)
        h = jnp.maximum(
            dot(w2_ref[...].astype(bf16), h.astype(bf16), dims) + b2_ref[...],
            0.0)
        o_ref[...] = dot(w3_ref[...], h, dims) + b3_ref[...]

    return pl.pallas_call(
        body,
        grid=(B // blk,),
        in_specs=[
            pl.BlockSpec((D2, blk), lambda i: (0, i)),
            pl.BlockSpec((128, D2), lambda i: (0, 0)),
            pl.BlockSpec((128, 1), lambda i: (0, 0)),
            pl.BlockSpec((64, 128), lambda i: (0, 0)),
            pl.BlockSpec((64, 1), lambda i: (0, 0)),
            pl.BlockSpec((1, 64), lambda i: (0, 0)),
            pl.BlockSpec((1, 1), lambda i: (0, 0)),
        ],
        out_specs=pl.BlockSpec((1, blk), lambda i: (0, i)),
        out_shape=jax.ShapeDtypeStruct((1, B), jnp.float32),
    )(xt, w1, b1, w2, b2, w3, b3)


def kernel(u, v, emb, W1, b1, W2, b2, W3, b3):
    u = u.astype(jnp.int32)
    v = v.astype(jnp.int32)
    xt = _gather_transposed(emb.T, u, v)
    out_t = _mlp_t(xt, W1, b1.reshape(-1, 1), W2, b2.reshape(-1, 1),
                   W3, b3.reshape(-1, 1))
    return out_t.reshape(-1, 1)


# speedup vs baseline: 1.0959x; 1.0121x over previous
import functools

import jax
import jax.numpy as jnp
from jax import lax
from jax.experimental import pallas as pl
from jax.experimental.pallas import tpu as pltpu
from jax.experimental.pallas import tpu_sc as plsc

_NC = 2
_NS = 16

_CHUNK = 4096


def _gather_transposed(emb_t, u, v):
    D, V = emb_t.shape
    B = u.shape[0]
    n_chunks = B // _CHUNK

    mesh = plsc.VectorSubcoreMesh(core_axis_name="c", subcore_axis_name="s")

    @functools.partial(
        pl.kernel,
        mesh=mesh,
        compiler_params=pltpu.CompilerParams(needs_layout_passes=False),
        out_type=jax.ShapeDtypeStruct((2 * D, B), jnp.float32),
        scratch_types=[
            pltpu.VMEM((V,), jnp.float32),
            pltpu.VMEM((_CHUNK,), jnp.int32),
            pltpu.VMEM((_CHUNK,), jnp.int32),
            pltpu.VMEM((_CHUNK,), jnp.int32),
            pltpu.VMEM((_CHUNK,), jnp.int32),
            pltpu.VMEM((_CHUNK,), jnp.float32),
            pltpu.VMEM((_CHUNK,), jnp.float32),
            pltpu.VMEM((_CHUNK,), jnp.float32),
            pltpu.SemaphoreType.DMA,
            pltpu.SemaphoreType.DMA,
            pltpu.SemaphoreType.DMA,
            pltpu.SemaphoreType.DMA,
            pltpu.SemaphoreType.DMA,
            pltpu.SemaphoreType.DMA,
            pltpu.SemaphoreType.DMA,
        ],
    )
    def gather_k(emb_hbm, u_hbm, v_hbm, xt_hbm, feat,
                 idxb0, idxb1, idxb2, idxb3, outb0, outb1, outb2,
                 si0, si1, si2, si3, so0, so1, so2):
        wid = lax.axis_index("s") * _NC + lax.axis_index("c")
        idxb = (idxb0, idxb1, idxb2, idxb3)
        outb = (outb0, outb1, outb2)
        si = (si0, si1, si2, si3)
        so = (so0, so1, so2)
        tasks = [(u_hbm, wid, c) for c in range(n_chunks)]
        tasks += [(v_hbm, wid + D, c) for c in range(n_chunks)]
        nt = len(tasks)

        def start_idx(t):
            idx_hbm, _, c = tasks[t]
            slot = t % 4
            return pltpu.async_copy(
                idx_hbm.at[pl.ds(c * _CHUNK, _CHUNK)], idxb[slot], si[slot])

        h_idx = {t: start_idx(t) for t in range(min(4, nt))}
        with jax.named_scope("feat_stage"):
            pltpu.sync_copy(emb_hbm.at[wid], feat)
        h_out = {}
        for t in range(nt):
            islot = t % 4
            oslot = t % 3
            h_idx[t].wait()
            if t + 4 < nt:
                h_idx[t + 4] = start_idx(t + 4)
            if t >= 3:
                h_out[t - 3].wait()
            src = idxb[islot]
            dst = outb[oslot]

            with jax.named_scope("gather"):
                @plsc.parallel_loop(0, _CHUNK // 16, unroll=16)
                def _(i):
                    iv = src[pl.ds(i * 16, 16)]
                    dst[pl.ds(i * 16, 16)] = plsc.load_gather(feat, [iv])

            _, row, c = tasks[t]
            h_out[t] = pltpu.async_copy(
                outb[oslot], xt_hbm.at[row, pl.ds(c * _CHUNK, _CHUNK)],
                so[oslot])
        h_out[nt - 3].wait()
        h_out[nt - 2].wait()
        h_out[nt - 1].wait()

    return gather_k(emb_t, u, v)


def _mlp_t(xt, w1, b1, w2, b2, w3, b3):
    D2, B = xt.shape
    blk = 8192
    dot = functools.partial(
        lax.dot_general, preferred_element_type=jnp.float32)
    dims = (((1,), (0,)), ((), ()))

    def body(xt_ref, w1_ref, b1_ref, w2_ref, b2_ref, w3_ref, b3_ref, o_ref):
        h = jnp.maximum(dot(w1_ref[...], xt_ref[...], dims) + b1_ref[...], 0.0)
        h = jnp.maximum(dot(w2_ref[...], h, dims) + b2_ref[...], 0.0)
        o_ref[...] = dot(w3_ref[...], h, dims) + b3_ref[...]

    return pl.pallas_call(
        body,
        grid=(B // blk,),
        in_specs=[
            pl.BlockSpec((D2, blk), lambda i: (0, i)),
            pl.BlockSpec((128, D2), lambda i: (0, 0)),
            pl.BlockSpec((128, 1), lambda i: (0, 0)),
            pl.BlockSpec((64, 128), lambda i: (0, 0)),
            pl.BlockSpec((64, 1), lambda i: (0, 0)),
            pl.BlockSpec((1, 64), lambda i: (0, 0)),
            pl.BlockSpec((1, 1), lambda i: (0, 0)),
        ],
        out_specs=pl.BlockSpec((1, blk), lambda i: (0, i)),
        out_shape=jax.ShapeDtypeStruct((1, B), jnp.float32),
    )(xt, w1, b1, w2, b2, w3, b3)


def kernel(u, v, emb, W1, b1, W2, b2, W3, b3):
    u = u.astype(jnp.int32)
    v = v.astype(jnp.int32)
    xt = _gather_transposed(emb.T, u, v)
    out_t = _mlp_t(xt, W1, b1.reshape(-1, 1), W2, b2.reshape(-1, 1),
                   W3, b3.reshape(-1, 1))
    return out_t.reshape(-1, 1)
